# baseline (speedup 1.0000x reference)
import functools

import jax
import jax.numpy as jnp
from jax.experimental import pallas as pl
from jax.experimental.pallas import tpu as pltpu

THETA_GLO = 1.15
DELTA = 5
GAMMA = 0.5
EPS = 1e-6

BLOCK_B = 1024
OH_S = 256.0
K_CAT = 256


def _loss_kernel(ya_ref, yp_ref, ypd_ref, ta_ref, t_ref, out_ref,
                 pcat_ref, *, d, n_rows):
    i = pl.program_id(0)

    @pl.when(i == 0)
    def _():
        p = yp_ref[...]
        t = t_ref[...]
        c_p = (jnp.sum(p * p - (2.0 * EPS) * p, axis=1, keepdims=True)
               + d * EPS * EPS)
        c_hi = c_p.astype(jnp.bfloat16).astype(jnp.float32)
        c_lo = c_p - c_hi
        iota = jax.lax.broadcasted_iota(jnp.int32, (p.shape[0], d), 1)
        oh_p = jnp.where(iota == t, OH_S, 0.0)
        oh_p = jnp.where(iota == 100, c_hi, oh_p)
        oh_p = jnp.where(iota == 101, c_lo, oh_p)
        pcat_ref[:, :d] = p.astype(jnp.bfloat16)
        pcat_ref[:, d:] = oh_p.astype(jnp.bfloat16)

    a = ya_ref[...]
    ta = ta_ref[...]
    iota_a = jax.lax.broadcasted_iota(jnp.int32, (a.shape[0], d), 1)
    oh_a = jnp.where(iota_a == ta, OH_S, 0.0)
    oh_a = jnp.where((iota_a == 100) | (iota_a == 101), 1.0, oh_a)
    a_cat = jnp.concatenate(
        [(-2.0 * a).astype(jnp.bfloat16), oh_a.astype(jnp.bfloat16)],
        axis=1)

    e_t = jax.lax.dot_general(
        pcat_ref[...], a_cat, (((1,), (1,)), ((), ())),
        preferred_element_type=jnp.float32)
    mv = jnp.min(e_t, axis=0, keepdims=True)

    ones_row = jnp.ones((1, d), jnp.float32)
    r_a = jax.lax.dot_general(
        ones_row, a * a + (2.0 * EPS) * a,
        (((1,), (1,)), ((), ())), preferred_element_type=jnp.float32)
    diff = a - ypd_ref[...] + EPS
    d_p2 = jax.lax.dot_general(
        ones_row, diff * diff,
        (((1,), (1,)), ((), ())), preferred_element_type=jnp.float32)

    d_n = jnp.sqrt(jnp.maximum(mv + r_a, 0.0))
    d_p = jnp.sqrt(jnp.maximum(d_p2, 0.0))

    theta = GAMMA * (d_p + d_n) * 0.5 + (1.0 - GAMMA) * THETA_GLO
    scale = 2.0 * DELTA
    loss = -(jax.nn.log_sigmoid(scale * (theta - d_p))
             + jax.nn.log_sigmoid(scale * (d_n - theta))) / scale

    @pl.when(i == 0)
    def _():
        out_ref[...] = jnp.zeros((1, 1), jnp.float32)

    out_ref[...] += jnp.sum(loss, keepdims=True) / n_rows


def kernel(y_a, y_p, targets):
    b, d = y_a.shape
    targets = targets.astype(jnp.int32)
    t_row = targets.reshape(b, 1)
    grid = b // BLOCK_B

    out = pl.pallas_call(
        functools.partial(_loss_kernel, d=d, n_rows=b),
        grid=(grid,),
        in_specs=[
            pl.BlockSpec((BLOCK_B, d), lambda i: (i, 0)),
            pl.BlockSpec((b, d), lambda i: (0, 0)),
            pl.BlockSpec((BLOCK_B, d), lambda i: (i, 0)),
            pl.BlockSpec((BLOCK_B, 1), lambda i: (i, 0)),
            pl.BlockSpec((b, 1), lambda i: (0, 0)),
        ],
        out_specs=pl.BlockSpec((1, 1), lambda i: (0, 0)),
        out_shape=jax.ShapeDtypeStruct((1, 1), jnp.float32),
        scratch_shapes=[
            pltpu.VMEM((b, K_CAT), jnp.bfloat16),
        ],
    )(y_a, y_p, y_p, t_row, t_row)

    return out[0, 0]


# speedup vs baseline: 1.0676x; 1.0676x over previous
import functools

import jax
import jax.numpy as jnp
from jax.experimental import pallas as pl
from jax.experimental.pallas import tpu as pltpu

THETA_GLO = 1.15
DELTA = 5
GAMMA = 0.5
EPS = 1e-6

BLOCK_B = 2048
OH_S = 256.0
K_CAT = 256


def _loss_kernel(ya_ref, yp_ref, t_ref, out_ref,
                 acat_ref, pcat_ref, ra_ref, dp2_ref, *, d, n_rows):
    i = pl.program_id(0)

    @pl.when(i == 0)
    def _():
        p = yp_ref[...]
        a_full = ya_ref[...]
        t = t_ref[...]
        c_p = (jnp.sum(p * p - (2.0 * EPS) * p, axis=1, keepdims=True)
               + d * EPS * EPS)
        c_hi = c_p.astype(jnp.bfloat16).astype(jnp.float32)
        c_lo = c_p - c_hi
        iota = jax.lax.broadcasted_iota(jnp.int32, (p.shape[0], d), 1)
        oh = jnp.where(iota == t, OH_S, 0.0)
        oh_p = jnp.where(iota == 100, c_hi, oh)
        oh_p = jnp.where(iota == 101, c_lo, oh_p)
        pcat_ref[:, :d] = p.astype(jnp.bfloat16)
        pcat_ref[:, d:] = oh_p.astype(jnp.bfloat16)
        oh_a = jnp.where((iota == 100) | (iota == 101), 1.0, oh)
        acat_ref[:, :d] = (-2.0 * a_full).astype(jnp.bfloat16)
        acat_ref[:, d:] = oh_a.astype(jnp.bfloat16)
        ones_row = jnp.ones((1, d), jnp.float32)
        ra_ref[...] = jax.lax.dot_general(
            ones_row, a_full * a_full + (2.0 * EPS) * a_full,
            (((1,), (1,)), ((), ())), preferred_element_type=jnp.float32)
        diff = a_full - p + EPS
        dp2_ref[...] = jax.lax.dot_general(
            ones_row, diff * diff,
            (((1,), (1,)), ((), ())), preferred_element_type=jnp.float32)

    a_cat = acat_ref[pl.ds(i * BLOCK_B, BLOCK_B), :]
    e_t = jax.lax.dot_general(
        pcat_ref[...], a_cat, (((1,), (1,)), ((), ())),
        preferred_element_type=jnp.float32)
    mv = jnp.min(e_t, axis=0, keepdims=True)

    r_a = ra_ref[:, pl.ds(i * BLOCK_B, BLOCK_B)]
    d_p2 = dp2_ref[:, pl.ds(i * BLOCK_B, BLOCK_B)]

    d_n = jnp.sqrt(jnp.maximum(mv + r_a, 0.0))
    d_p = jnp.sqrt(jnp.maximum(d_p2, 0.0))

    theta = GAMMA * (d_p + d_n) * 0.5 + (1.0 - GAMMA) * THETA_GLO
    scale = 2.0 * DELTA
    loss = -(jax.nn.log_sigmoid(scale * (theta - d_p))
             + jax.nn.log_sigmoid(scale * (d_n - theta))) / scale

    @pl.when(i == 0)
    def _():
        out_ref[...] = jnp.zeros((1, 1), jnp.float32)

    out_ref[...] += jnp.sum(loss, keepdims=True) / n_rows


def kernel(y_a, y_p, targets):
    b, d = y_a.shape
    targets = targets.astype(jnp.int32)
    t_row = targets.reshape(b, 1)
    grid = b // BLOCK_B

    out = pl.pallas_call(
        functools.partial(_loss_kernel, d=d, n_rows=b),
        grid=(grid,),
        in_specs=[
            pl.BlockSpec((b, d), lambda i: (0, 0)),
            pl.BlockSpec((b, d), lambda i: (0, 0)),
            pl.BlockSpec((b, 1), lambda i: (0, 0)),
        ],
        out_specs=pl.BlockSpec((1, 1), lambda i: (0, 0)),
        out_shape=jax.ShapeDtypeStruct((1, 1), jnp.float32),
        scratch_shapes=[
            pltpu.VMEM((b, K_CAT), jnp.bfloat16),
            pltpu.VMEM((b, K_CAT), jnp.bfloat16),
            pltpu.VMEM((1, b), jnp.float32),
            pltpu.VMEM((1, b), jnp.float32),
        ],
    )(y_a, y_p, t_row)

    return out[0, 0]


# speedup vs baseline: 1.1335x; 1.0617x over previous
import functools

import jax
import jax.numpy as jnp
from jax.experimental import pallas as pl
from jax.experimental.pallas import tpu as pltpu

THETA_GLO = 1.15
DELTA = 5
GAMMA = 0.5
EPS = 1e-6

BLOCK_B = 1024
OH_S = 256.0
K_CAT = 256


def _loss_kernel(ya_ref, yp_ref, t_ref, out_ref,
                 acat_ref, pcat_ref, ra_ref, dp2_ref, *, d, n_rows):
    i = pl.program_id(0)

    @pl.when(i == 0)
    def _():
        p = yp_ref[...]
        a_full = ya_ref[...]
        t = t_ref[...]
        c_p = (jnp.sum(p * p - (2.0 * EPS) * p, axis=1, keepdims=True)
               + d * EPS * EPS)
        c_hi = c_p.astype(jnp.bfloat16).astype(jnp.float32)
        c_lo = c_p - c_hi
        iota = jax.lax.broadcasted_iota(jnp.int32, (p.shape[0], d), 1)
        oh = jnp.where(iota == t, OH_S, 0.0)
        oh_p = jnp.where(iota == 100, c_hi, oh)
        oh_p = jnp.where(iota == 101, c_lo, oh_p)
        pcat_ref[:, :d] = p.astype(jnp.bfloat16)
        pcat_ref[:, d:] = oh_p.astype(jnp.bfloat16)
        oh_a = jnp.where((iota == 100) | (iota == 101), 1.0, oh)
        acat_ref[:, :d] = (-2.0 * a_full).astype(jnp.bfloat16)
        acat_ref[:, d:] = oh_a.astype(jnp.bfloat16)
        ones_row = jnp.ones((1, d), jnp.float32)
        ra_ref[...] = jax.lax.dot_general(
            ones_row, a_full * a_full + (2.0 * EPS) * a_full,
            (((1,), (1,)), ((), ())), preferred_element_type=jnp.float32)
        diff = a_full - p + EPS
        dp2_ref[...] = jax.lax.dot_general(
            ones_row, diff * diff,
            (((1,), (1,)), ((), ())), preferred_element_type=jnp.float32)

    del i
    b = pcat_ref.shape[0]
    acc = None
    for jh in range(b // BLOCK_B):
        a_cat = acat_ref[pl.ds(jh * BLOCK_B, BLOCK_B), :]
        e_t = jax.lax.dot_general(
            pcat_ref[...], a_cat, (((1,), (1,)), ((), ())),
            preferred_element_type=jnp.float32)
        mv = jnp.min(e_t, axis=0, keepdims=True)

        r_a = ra_ref[:, pl.ds(jh * BLOCK_B, BLOCK_B)]
        d_p2 = dp2_ref[:, pl.ds(jh * BLOCK_B, BLOCK_B)]

        d_n = jnp.sqrt(jnp.maximum(mv + r_a, 0.0))
        d_p = jnp.sqrt(jnp.maximum(d_p2, 0.0))

        theta = GAMMA * (d_p + d_n) * 0.5 + (1.0 - GAMMA) * THETA_GLO
        scale = 2.0 * DELTA
        loss = -(jax.nn.log_sigmoid(scale * (theta - d_p))
                 + jax.nn.log_sigmoid(scale * (d_n - theta))) / scale
        s = jnp.sum(loss, keepdims=True)
        acc = s if acc is None else acc + s

    out_ref[...] = acc / n_rows


def kernel(y_a, y_p, targets):
    b, d = y_a.shape
    targets = targets.astype(jnp.int32)
    t_row = targets.reshape(b, 1)
    grid = 1

    out = pl.pallas_call(
        functools.partial(_loss_kernel, d=d, n_rows=b),
        grid=(grid,),
        in_specs=[
            pl.BlockSpec((b, d), lambda i: (0, 0)),
            pl.BlockSpec((b, d), lambda i: (0, 0)),
            pl.BlockSpec((b, 1), lambda i: (0, 0)),
        ],
        out_specs=pl.BlockSpec((1, 1), lambda i: (0, 0)),
        out_shape=jax.ShapeDtypeStruct((1, 1), jnp.float32),
        scratch_shapes=[
            pltpu.VMEM((b, K_CAT), jnp.bfloat16),
            pltpu.VMEM((b, K_CAT), jnp.bfloat16),
            pltpu.VMEM((1, b), jnp.float32),
            pltpu.VMEM((1, b), jnp.float32),
        ],
    )(y_a, y_p, t_row)

    return out[0, 0]
